# baseline (speedup 1.0000x reference)
import functools

import jax
import jax.numpy as jnp
from jax import lax
from jax.experimental import pallas as pl
from jax.experimental.pallas import tpu as pltpu
from jax.experimental.pallas import tpu_sc as plsc

_NC = 2
_NS = 16
_NW = _NC * _NS


@functools.partial(jax.jit, static_argnames=("C",))
def _gather(idx, table, C):
    B, = idx.shape
    V, D = table.shape
    b_per_w = B // _NW
    n_chunks = b_per_w // C
    assert b_per_w % C == 0 and n_chunks % 2 == 0

    Dout = 64

    mesh = plsc.VectorSubcoreMesh(core_axis_name="c", subcore_axis_name="s")

    @functools.partial(
        pl.kernel,
        mesh=mesh,
        compiler_params=pltpu.CompilerParams(use_tc_tiling_on_sc=True),
        out_type=jax.ShapeDtypeStruct((B, D), jnp.float32),
        scratch_types=[
            pltpu.VMEM((b_per_w,), jnp.int32),
            pltpu.VMEM((2, C, D), jnp.float32),
            pltpu.SemaphoreType.DMA,
            pltpu.SemaphoreType.DMA,
        ],
    )
    def k(idx_hbm, table_hbm, out_hbm, idx_v, rows_v, gsem0, gsem1):
        wid = lax.axis_index("s") * _NC + lax.axis_index("c")
        base = wid * b_per_w
        pltpu.sync_copy(idx_hbm.at[pl.ds(base, b_per_w)], idx_v)
        gsems = (gsem0, gsem1)

        def fire(chunk, slot):
            pltpu.async_copy(
                table_hbm.at[idx_v.at[pl.ds(chunk * C, C)]],
                rows_v.at[slot],
                gsems[slot],
            )

        def wait_writeback(chunk, slot):
            pltpu.make_async_copy(
                table_hbm.at[idx_v.at[pl.ds(chunk * C, C)]],
                rows_v.at[slot],
                gsems[slot],
            ).wait()
            pltpu.sync_copy(
                rows_v.at[slot], out_hbm.at[pl.ds(base + chunk * C, C)]
            )

        fire(0, 0)
        fire(1, 1)

        @pl.loop(0, n_chunks, step=2)
        def _(i):
            for b in range(2):
                chunk = i + b
                wait_writeback(chunk, b)

                @pl.when(chunk + 2 < n_chunks)
                def _():
                    fire(chunk + 2, b)

    return k(idx, table)


def kernel(inputs, table):
    B0, B1 = inputs.shape
    _, D = table.shape
    idx = inputs.reshape(B0 * B1).astype(jnp.int32)
    tpad = jnp.pad(table, ((0, 0), (0, D)))
    out = _gather(idx, tpad, C=256)
    return out[:, :D].reshape(B0, B1, D)


# speedup vs baseline: 1.0634x; 1.0634x over previous
import functools

import jax
import jax.numpy as jnp
from jax import lax
from jax.experimental import pallas as pl
from jax.experimental.pallas import tpu as pltpu
from jax.experimental.pallas import tpu_sc as plsc

_NC = 2
_NS = 16
_NW = _NC * _NS


@functools.partial(jax.jit, static_argnames=("C",))
def _gather(idx, table, C):
    B, = idx.shape
    V, D = table.shape
    b_per_w = B // _NW
    n_chunks = b_per_w // C
    assert b_per_w % C == 0 and n_chunks % 2 == 0

    mesh = plsc.VectorSubcoreMesh(core_axis_name="c", subcore_axis_name="s")

    @functools.partial(
        pl.kernel,
        mesh=mesh,
        compiler_params=pltpu.CompilerParams(use_tc_tiling_on_sc=False),
        out_type=jax.ShapeDtypeStruct((B, D), jnp.float32),
        scratch_types=[
            pltpu.VMEM((b_per_w,), jnp.int32),
            pltpu.VMEM((2, C, D), jnp.float32),
            pltpu.SemaphoreType.DMA,
            pltpu.SemaphoreType.DMA,
        ],
    )
    def k(idx_hbm, table_hbm, out_hbm, idx_v, rows_v, gsem0, gsem1):
        wid = lax.axis_index("s") * _NC + lax.axis_index("c")
        base = wid * b_per_w
        pltpu.sync_copy(idx_hbm.at[pl.ds(base, b_per_w)], idx_v)
        gsems = (gsem0, gsem1)

        def fire(chunk, slot):
            pltpu.async_copy(
                table_hbm.at[idx_v.at[pl.ds(chunk * C, C)]],
                rows_v.at[slot],
                gsems[slot],
            )

        def wait_writeback(chunk, slot):
            pltpu.make_async_copy(
                table_hbm.at[idx_v.at[pl.ds(chunk * C, C)]],
                rows_v.at[slot],
                gsems[slot],
            ).wait()
            pltpu.sync_copy(
                rows_v.at[slot], out_hbm.at[pl.ds(base + chunk * C, C)]
            )

        fire(0, 0)
        fire(1, 1)

        @pl.loop(0, n_chunks, step=2)
        def _(i):
            for b in range(2):
                chunk = i + b
                wait_writeback(chunk, b)

                @pl.when(chunk + 2 < n_chunks)
                def _():
                    fire(chunk + 2, b)

    return k(idx, table)


def kernel(inputs, table):
    B0, B1 = inputs.shape
    _, D = table.shape
    idx = inputs.reshape(B0 * B1).astype(jnp.int32)
    out = _gather(idx, table, C=512)
    return out.reshape(B0, B1, D)
